# baseline (speedup 1.0000x reference)
import functools

import numpy as np

import jax
import jax.numpy as jnp
from jax import lax
from jax.experimental import pallas as pl
from jax.experimental.pallas import tpu as pltpu
from jax.experimental.pallas import tpu_sc as plsc

_NC = 2
_NS = 16
_NW = _NC * _NS
_K = 10


def _edge_pass(h, src2d, dst2d, ew2d, zeros, n_pad, nrw0, nrw1):
    f = h.shape[1]
    nrw_max = max(nrw0, nrw1)
    rows_per_sub = n_pad // _NS
    mesh = plsc.VectorSubcoreMesh(core_axis_name="c", subcore_axis_name="s")

    @functools.partial(
        pl.kernel,
        out_type=jax.ShapeDtypeStruct((_NC, n_pad, f), jnp.float32),
        mesh=mesh,
        scratch_types=[
            pltpu.VMEM((nrw_max * 128,), jnp.int32),
            pltpu.VMEM((nrw_max, 128), jnp.int32),
            pltpu.VMEM((nrw_max * 128,), jnp.float32),
            pltpu.VMEM((_K * 128, f), jnp.float32),
            pltpu.VMEM((_K * 128, f), jnp.float32),
            pltpu.VMEM_SHARED((n_pad, f), jnp.float32),
            pltpu.VMEM_SHARED(h.shape, jnp.float32),
            pltpu.SemaphoreType.DMA,
            pltpu.SemaphoreType.DMA,
            pltpu.SemaphoreType.DMA,
        ],
        compiler_params=pltpu.CompilerParams(use_tc_tiling_on_sc=False),
    )
    def kern(h_hbm, src_hbm, dst_hbm, ew_hbm, z_hbm, out_hbm,
             src_v, dst_v, ew_v, rows_a, rows_b, acc_sh, h_sh,
             sem_a, sem_b, sem_s):
        cid = lax.axis_index("c")
        sid = lax.axis_index("s")
        nb = sid * rows_per_sub

        @pl.when((cid != 0) & (sid == 0))
        def _():
            pltpu.sync_copy(h_hbm, h_sh)

        pltpu.sync_copy(z_hbm.at[pl.ds(nb, rows_per_sub)],
                        acc_sh.at[pl.ds(nb, rows_per_sub)])
        plsc.subcore_barrier()

        def issue(ci, buf, sem, table):
            r0 = ci * _K
            for j in range(_K):
                pltpu.async_copy(table.at[src_v.at[pl.ds((r0 + j) * 128, 128)]],
                                 buf.at[pl.ds(j * 128, 128)], sem)

        def wait(ci, buf, sem, table):
            r0 = ci * _K
            for j in range(_K):
                pltpu.make_async_copy(
                    table.at[src_v.at[pl.ds((r0 + j) * 128, 128)]],
                    buf.at[pl.ds(j * 128, 128)], sem).wait()

        def mult_scatter(ci, buf):
            r0 = ci * _K

            @pl.loop(0, _K)
            def _(j):
                @plsc.parallel_loop(0, 8)
                def _(l16):
                    wvec = ew_v[pl.ds((r0 + j) * 128 + l16 * 16, 16)]
                    base_e = j * 128 + l16 * 16
                    for t in range(16):
                        w = wvec[t]
                        buf[base_e + t, :] = buf[base_e + t, :] * w

                pltpu.async_copy(buf.at[pl.ds(j * 128, 128)],
                                 acc_sh.at[dst_v.at[r0 + j]], sem_s, add=True)

            for j in range(_K):
                pltpu.make_async_copy(buf.at[pl.ds(j * 128, 128)],
                                      acc_sh.at[dst_v.at[r0 + j]], sem_s).wait()

        def run(nrw_c, row_base, table):
            chunks = nrw_c // _K
            pltpu.sync_copy(src_hbm.at[pl.ds(row_base * 128, nrw_c * 128)],
                            src_v.at[pl.ds(0, nrw_c * 128)])
            pltpu.sync_copy(dst_hbm.at[pl.ds(row_base, nrw_c)],
                            dst_v.at[pl.ds(0, nrw_c)])
            pltpu.sync_copy(ew_hbm.at[pl.ds(row_base * 128, nrw_c * 128)],
                            ew_v.at[pl.ds(0, nrw_c * 128)])
            issue(0, rows_a, sem_a, table)
            issue(1, rows_b, sem_b, table)

            @pl.loop(0, chunks // 2 - 1)
            def _(cp):
                ci = cp * 2
                wait(ci, rows_a, sem_a, table)
                mult_scatter(ci, rows_a)
                issue(ci + 2, rows_a, sem_a, table)
                wait(ci + 1, rows_b, sem_b, table)
                mult_scatter(ci + 1, rows_b)
                issue(ci + 3, rows_b, sem_b, table)

            wait(chunks - 2, rows_a, sem_a, table)
            mult_scatter(chunks - 2, rows_a)
            wait(chunks - 1, rows_b, sem_b, table)
            mult_scatter(chunks - 1, rows_b)

        @pl.when(cid == 0)
        def _():
            run(nrw0, sid * nrw0, h_hbm)

        @pl.when(cid != 0)
        def _():
            run(nrw1, _NS * nrw0 + sid * nrw1, h_sh)

        plsc.subcore_barrier()
        pltpu.sync_copy(acc_sh.at[pl.ds(nb, rows_per_sub)],
                        out_hbm.at[cid, pl.ds(nb, rows_per_sub)])

    return kern(h, src2d, dst2d, ew2d, zeros)


def _tc_matmul(x, w):
    n = x.shape[0]
    f = w.shape[1]

    def body(x_ref, w_ref, o_ref):
        o_ref[...] = jnp.dot(x_ref[...], w_ref[...],
                             preferred_element_type=jnp.float32)

    return pl.pallas_call(
        body, out_shape=jax.ShapeDtypeStruct((n, f), jnp.float32))(x, w)


def _tc_layer2_blocked(p_blocked, b1_tile, w2bd):
    nb8 = p_blocked.shape[1]

    def body(p_ref, b_ref, w_ref, o_ref):
        q = p_ref[0] + p_ref[1] + b_ref[...]
        z = jnp.maximum(q, 0.0)
        o_ref[...] = jnp.dot(z, w_ref[...], preferred_element_type=jnp.float32)

    return pl.pallas_call(
        body, out_shape=jax.ShapeDtypeStruct((nb8, 128), jnp.float32))(
            p_blocked, b1_tile, w2bd)


def _tc_final_blocked(p_blocked, b2_tile, f, c):
    nb8 = p_blocked.shape[1]
    g = 128 // f
    lane = np.arange(128)
    valid = (lane % f) < c
    neg = np.where(valid, 0.0, -1e30).astype(np.float32).reshape(1, 128)
    sel = np.where(lane % f == 0, 1.0, 0.0).astype(np.float32).reshape(1, 128)
    bmat = np.zeros((128, 128), np.float32)
    bmat[(lane // f) * f, lane] = 1.0
    gmat = np.zeros((128, 128), np.float32)
    gmat[np.equal.outer(lane // f, lane // f)] = 1.0
    assert g * f == 128

    def body(p_ref, b_ref, neg_ref, sel_ref, bm_ref, gm_ref, o_ref):
        s = p_ref[0] + p_ref[1] + b_ref[...]
        sm = s + neg_ref[...]
        m = sm
        for sh in (1, 2, 4, 8):
            m = jnp.maximum(m, jnp.roll(m, -sh, axis=1))
        mb = jnp.dot(m * sel_ref[...], bm_ref[...],
                     preferred_element_type=jnp.float32,
                     precision=jax.lax.Precision.HIGHEST)
        e = jnp.exp(jnp.minimum(sm - mb, 0.0))
        se = jnp.dot(e, gm_ref[...], preferred_element_type=jnp.float32,
                     precision=jax.lax.Precision.HIGHEST)
        o_ref[...] = s - mb - jnp.log(se)

    return pl.pallas_call(
        body, out_shape=jax.ShapeDtypeStruct((nb8, 128), jnp.float32))(
            p_blocked, b2_tile, jnp.asarray(neg), jnp.asarray(sel),
            jnp.asarray(bmat), jnp.asarray(gmat))


def kernel(x, edge_index, edge_weight, W1, b1, W2, b2):
    n = x.shape[0]
    e = edge_weight.shape[0]
    h_dim = W1.shape[1]
    c = W2.shape[1]

    row_quantum = _NS * 160
    rows_total = -(-(-(-e // 128)) // row_quantum) * row_quantum
    per_sub = rows_total // _NS
    nrw0 = per_sub * 4 // 8
    nrw1 = per_sub * 4 // 8
    ep = rows_total * 128
    n_pad = -(-n // (_NS * 8)) * (_NS * 8)
    pad = ep - e
    pad_iota = np.arange(pad, dtype=np.int32)
    src = jnp.concatenate(
        [edge_index[0].astype(jnp.int32),
         jnp.asarray((pad_iota * 16) % n, jnp.int32)])
    dst = jnp.concatenate(
        [edge_index[1].astype(jnp.int32),
         jnp.asarray((pad_iota * 16) % n_pad, jnp.int32)]
    ).reshape(-1, 128)
    ew = jnp.concatenate(
        [edge_weight.astype(jnp.float32),
         jnp.zeros((pad,), jnp.float32)])
    zeros = jnp.zeros((n_pad, h_dim), jnp.float32)
    w2p = jnp.zeros((h_dim, h_dim), jnp.float32).at[:, :c].set(W2)

    b1_tile = jnp.tile(b1, 8).reshape(1, 128)
    w2bd = jnp.kron(jnp.eye(8, dtype=jnp.float32), w2p)

    h1 = _tc_matmul(x, W1)
    p1 = _edge_pass(h1, src, dst, ew, zeros, n_pad, nrw0, nrw1)
    zb = _tc_layer2_blocked(p1.reshape(_NC, n_pad // 8, 128), b1_tile, w2bd)
    z = zb.reshape(n_pad, h_dim)
    p2 = _edge_pass(z, src, dst, ew, zeros, n_pad, nrw0, nrw1)
    b2_tile = jnp.tile(
        jnp.concatenate([b2, jnp.zeros((h_dim - c,), jnp.float32)]),
        128 // h_dim).reshape(1, 128)
    ob = _tc_final_blocked(p2.reshape(_NC, n_pad // 8, 128), b2_tile,
                           h_dim, c)
    return ob.reshape(n_pad, h_dim)[:n, :c]


# speedup vs baseline: 1.0757x; 1.0757x over previous
import functools

import numpy as np

import jax
import jax.numpy as jnp
from jax import lax
from jax.experimental import pallas as pl
from jax.experimental.pallas import tpu as pltpu
from jax.experimental.pallas import tpu_sc as plsc

_NC = 2
_NS = 16
_NW = _NC * _NS
_K = 10


def _edge_pass(h, src2d, dst2d, ew2d, zeros, n_pad, nrw0, nrw1):
    f = h.shape[1]
    nrw_max = max(nrw0, nrw1)
    rows_per_sub = n_pad // _NS
    mesh = plsc.VectorSubcoreMesh(core_axis_name="c", subcore_axis_name="s")

    @functools.partial(
        pl.kernel,
        out_type=jax.ShapeDtypeStruct((_NC, n_pad, f), jnp.float32),
        mesh=mesh,
        scratch_types=[
            pltpu.VMEM((nrw_max * 128,), jnp.int32),
            pltpu.VMEM((nrw_max, 128), jnp.int32),
            pltpu.VMEM((nrw_max * 128,), jnp.float32),
            pltpu.VMEM((_K * 128, f), jnp.float32),
            pltpu.VMEM((_K * 128, f), jnp.float32),
            pltpu.VMEM_SHARED((n_pad, f), jnp.float32),
            pltpu.VMEM_SHARED(h.shape, jnp.float32),
            pltpu.SemaphoreType.DMA,
            pltpu.SemaphoreType.DMA,
            pltpu.SemaphoreType.DMA,
        ],
        compiler_params=pltpu.CompilerParams(use_tc_tiling_on_sc=False),
    )
    def kern(h_hbm, src_hbm, dst_hbm, ew_hbm, z_hbm, out_hbm,
             src_v, dst_v, ew_v, rows_a, rows_b, acc_sh, h_sh,
             sem_a, sem_b, sem_s):
        cid = lax.axis_index("c")
        sid = lax.axis_index("s")
        nb = sid * rows_per_sub

        @pl.when((cid != 0) & (sid == 0))
        def _():
            pltpu.sync_copy(h_hbm, h_sh)

        pltpu.sync_copy(z_hbm.at[pl.ds(nb, rows_per_sub)],
                        acc_sh.at[pl.ds(nb, rows_per_sub)])
        plsc.subcore_barrier()

        def issue(ci, buf, sem, table):
            r0 = ci * _K
            for j in range(_K):
                pltpu.async_copy(table.at[src_v.at[pl.ds((r0 + j) * 128, 128)]],
                                 buf.at[pl.ds(j * 128, 128)], sem)

        def wait(ci, buf, sem, table):
            r0 = ci * _K
            for j in range(_K):
                pltpu.make_async_copy(
                    table.at[src_v.at[pl.ds((r0 + j) * 128, 128)]],
                    buf.at[pl.ds(j * 128, 128)], sem).wait()

        def mult_scatter(ci, buf):
            r0 = ci * _K

            @pl.loop(0, _K)
            def _(j):
                @plsc.parallel_loop(0, 8)
                def _(l16):
                    wvec = ew_v[pl.ds((r0 + j) * 128 + l16 * 16, 16)]
                    base_e = j * 128 + l16 * 16
                    for t in range(16):
                        w = wvec[t]
                        buf[base_e + t, :] = buf[base_e + t, :] * w

                pltpu.async_copy(buf.at[pl.ds(j * 128, 128)],
                                 acc_sh.at[dst_v.at[r0 + j]], sem_s, add=True)

            for j in range(_K):
                pltpu.make_async_copy(buf.at[pl.ds(j * 128, 128)],
                                      acc_sh.at[dst_v.at[r0 + j]], sem_s).wait()

        def run(nrw_c, row_base, table):
            chunks = nrw_c // _K
            pltpu.sync_copy(src_hbm.at[pl.ds(row_base * 128, nrw_c * 128)],
                            src_v.at[pl.ds(0, nrw_c * 128)])
            pltpu.sync_copy(dst_hbm.at[pl.ds(row_base, nrw_c)],
                            dst_v.at[pl.ds(0, nrw_c)])
            pltpu.sync_copy(ew_hbm.at[pl.ds(row_base * 128, nrw_c * 128)],
                            ew_v.at[pl.ds(0, nrw_c * 128)])
            issue(0, rows_a, sem_a, table)
            issue(1, rows_b, sem_b, table)

            @pl.loop(0, chunks // 2 - 1)
            def _(cp):
                ci = cp * 2
                wait(ci, rows_a, sem_a, table)
                mult_scatter(ci, rows_a)
                issue(ci + 2, rows_a, sem_a, table)
                wait(ci + 1, rows_b, sem_b, table)
                mult_scatter(ci + 1, rows_b)
                issue(ci + 3, rows_b, sem_b, table)

            wait(chunks - 2, rows_a, sem_a, table)
            mult_scatter(chunks - 2, rows_a)
            wait(chunks - 1, rows_b, sem_b, table)
            mult_scatter(chunks - 1, rows_b)

        @pl.when(cid == 0)
        def _():
            run(nrw0, sid * nrw0, h_hbm)

        @pl.when(cid != 0)
        def _():
            run(nrw1, _NS * nrw0 + sid * nrw1, h_sh)

        plsc.subcore_barrier()
        pltpu.sync_copy(acc_sh.at[pl.ds(nb, rows_per_sub)],
                        out_hbm.at[cid, pl.ds(nb, rows_per_sub)])

    return kern(h, src2d, dst2d, ew2d, zeros)


def _tc_matmul(x, w):
    n = x.shape[0]
    f = w.shape[1]

    def body(x_ref, w_ref, o_ref):
        o_ref[...] = jnp.dot(x_ref[...], w_ref[...],
                             preferred_element_type=jnp.float32)

    return pl.pallas_call(
        body, out_shape=jax.ShapeDtypeStruct((n, f), jnp.float32))(x, w)


def _tc_layer2_blocked(p_blocked, b1_tile, w2bd):
    nb8 = p_blocked.shape[1]

    def body(p_ref, b_ref, w_ref, o_ref):
        q = p_ref[0] + p_ref[1] + b_ref[...]
        z = jnp.maximum(q, 0.0)
        o_ref[...] = jnp.dot(z, w_ref[...], preferred_element_type=jnp.float32)

    return pl.pallas_call(
        body, out_shape=jax.ShapeDtypeStruct((nb8, 128), jnp.float32))(
            p_blocked, b1_tile, w2bd)


def _tc_final_blocked(p_blocked, b2_tile, f, c):
    nb8 = p_blocked.shape[1]
    g = 128 // f
    lane = np.arange(128)
    valid = (lane % f) < c
    neg = np.where(valid, 0.0, -1e30).astype(np.float32).reshape(1, 128)
    sel = np.where(lane % f == 0, 1.0, 0.0).astype(np.float32).reshape(1, 128)
    bmat = np.zeros((128, 128), np.float32)
    bmat[(lane // f) * f, lane] = 1.0
    gmat = np.zeros((128, 128), np.float32)
    gmat[np.equal.outer(lane // f, lane // f)] = 1.0
    assert g * f == 128

    def body(p_ref, b_ref, neg_ref, sel_ref, bm_ref, gm_ref, o_ref):
        s = p_ref[0] + p_ref[1] + b_ref[...]
        sm = s + neg_ref[...]
        m = sm
        for sh in (1, 2, 4, 8):
            m = jnp.maximum(m, jnp.roll(m, -sh, axis=1))
        mb = jnp.dot(m * sel_ref[...], bm_ref[...],
                     preferred_element_type=jnp.float32,
                     precision=jax.lax.Precision.HIGHEST)
        e = jnp.exp(jnp.minimum(sm - mb, 0.0))
        se = jnp.dot(e, gm_ref[...], preferred_element_type=jnp.float32,
                     precision=jax.lax.Precision.HIGHEST)
        o_ref[...] = s - mb - jnp.log(se)

    return pl.pallas_call(
        body, out_shape=jax.ShapeDtypeStruct((nb8, 128), jnp.float32))(
            p_blocked, b2_tile, jnp.asarray(neg), jnp.asarray(sel),
            jnp.asarray(bmat), jnp.asarray(gmat))


def kernel(x, edge_index, edge_weight, W1, b1, W2, b2):
    n = x.shape[0]
    e = edge_weight.shape[0]
    h_dim = W1.shape[1]
    c = W2.shape[1]

    row_quantum = _NS * 160
    rows_total = -(-(-(-e // 128)) // row_quantum) * row_quantum
    per_sub = rows_total // _NS
    nrw0 = per_sub * 5 // 8
    nrw1 = per_sub * 3 // 8
    ep = rows_total * 128
    n_pad = -(-n // (_NS * 8)) * (_NS * 8)
    pad = ep - e
    pad_iota = np.arange(pad, dtype=np.int32)
    src = jnp.concatenate(
        [edge_index[0].astype(jnp.int32),
         jnp.asarray((pad_iota * 16) % n, jnp.int32)])
    dst = jnp.concatenate(
        [edge_index[1].astype(jnp.int32),
         jnp.asarray((pad_iota * 16) % n_pad, jnp.int32)]
    ).reshape(-1, 128)
    ew = jnp.concatenate(
        [edge_weight.astype(jnp.float32),
         jnp.zeros((pad,), jnp.float32)])
    zeros = jnp.zeros((n_pad, h_dim), jnp.float32)
    w2p = jnp.zeros((h_dim, h_dim), jnp.float32).at[:, :c].set(W2)

    b1_tile = jnp.tile(b1, 8).reshape(1, 128)
    w2bd = jnp.kron(jnp.eye(8, dtype=jnp.float32), w2p)

    h1 = _tc_matmul(x, W1)
    p1 = _edge_pass(h1, src, dst, ew, zeros, n_pad, nrw0, nrw1)
    zb = _tc_layer2_blocked(p1.reshape(_NC, n_pad // 8, 128), b1_tile, w2bd)
    z = zb.reshape(n_pad, h_dim)
    p2 = _edge_pass(z, src, dst, ew, zeros, n_pad, nrw0, nrw1)
    b2_tile = jnp.tile(
        jnp.concatenate([b2, jnp.zeros((h_dim - c,), jnp.float32)]),
        128 // h_dim).reshape(1, 128)
    ob = _tc_final_blocked(p2.reshape(_NC, n_pad // 8, 128), b2_tile,
                           h_dim, c)
    return ob.reshape(n_pad, h_dim)[:n, :c]
